# Initial kernel scaffold; baseline (speedup 1.0000x reference)
import jax
import jax.numpy as jnp
from jax.experimental import pallas as pl


def kernel(x, edge_index, batch, W1, b1, W2, b2, Wf1, bf1, Wf2, bf2, Wf3, bf3):
    raise NotImplementedError("write your pallas kernel here")


# speedup vs baseline: 20.0772x; 20.0772x over previous
import jax
import jax.numpy as jnp
from jax import lax
from jax.experimental import pallas as pl
from jax.experimental.pallas import tpu as pltpu
from jax.experimental.pallas import tpu_sc as plsc

f32 = jnp.float32

N = 100000
NP = 100096
E = 1600000
G = 2048
NC = 2
NS = 16
NW = NC * NS
EC = E // NW
ECH = 1000
ENCH = EC // ECH
NRW = NP // NS
GRW = G // NS
BCH = N // ECH
PCH = 1000
PNCH = N // PCH

BB = 3128
NB = NP // BB


def _sc_deg_cnt(dst_h, batch_h, ones_h, zeros_h, degp_h, cntp_h,
                idx_v, ones_v, dacc, cacc):
    c = lax.axis_index("c")
    s = lax.axis_index("s")
    w = s * NC + c
    pltpu.sync_copy(zeros_h.at[pl.ds(s * NRW, NRW)], dacc.at[pl.ds(s * NRW, NRW)])
    pltpu.sync_copy(zeros_h.at[pl.ds(s * GRW, GRW)], cacc.at[pl.ds(s * GRW, GRW)])
    pltpu.sync_copy(ones_h, ones_v)
    plsc.subcore_barrier()

    @pl.loop(0, ENCH)
    def _(k):
        pltpu.sync_copy(dst_h.at[pl.ds(w * EC + k * ECH, ECH)], idx_v)
        pltpu.sync_copy(ones_v, dacc.at[idx_v], add=True)

    ncnt = jnp.where(w < BCH - 3 * NW, 4, 3)

    @pl.loop(0, ncnt)
    def _(t):
        pltpu.sync_copy(batch_h.at[pl.ds((w + NW * t) * ECH, ECH)], idx_v)
        pltpu.sync_copy(ones_v, cacc.at[idx_v], add=True)

    plsc.subcore_barrier()
    pltpu.sync_copy(dacc.at[pl.ds(s * NRW, NRW)], degp_h.at[c, pl.ds(s * NRW, NRW)])
    pltpu.sync_copy(cacc.at[pl.ds(s * GRW, GRW)], cntp_h.at[c, pl.ds(s * GRW, GRW)])


def _sc_agg1(src_h, dst_h, xsp_h, zeros_h, esum1p_h,
             sidx, didx, rows, acc, sem):
    c = lax.axis_index("c")
    s = lax.axis_index("s")
    w = s * NC + c
    pltpu.sync_copy(zeros_h.at[pl.ds(s * NRW, NRW)], acc.at[pl.ds(s * NRW, NRW)])
    plsc.subcore_barrier()

    @pl.loop(0, ENCH)
    def _(k):
        base = w * EC + k * ECH
        pltpu.sync_copy(src_h.at[pl.ds(base, ECH)], sidx)
        pltpu.sync_copy(dst_h.at[pl.ds(base, ECH)], didx)
        pltpu.async_copy(xsp_h.at[sidx], rows, sem).wait()
        pltpu.sync_copy(rows, acc.at[didx], add=True)

    plsc.subcore_barrier()
    pltpu.sync_copy(acc.at[pl.ds(s * NRW, NRW)], esum1p_h.at[c, pl.ds(s * NRW, NRW)])


def _sc_agg2(src_h, dst_h, z0, z1, z2, z3, zeros_h, esum2p_h,
             sidx, didx, rows, acc, sem):
    c = lax.axis_index("c")
    s = lax.axis_index("s")
    w = s * NC + c
    for ci, zt in enumerate((z0, z1, z2, z3)):
        pltpu.sync_copy(zeros_h.at[pl.ds(s * NRW, NRW)], acc.at[pl.ds(s * NRW, NRW)])
        plsc.subcore_barrier()

        @pl.loop(0, ENCH)
        def _(k):
            base = w * EC + k * ECH
            pltpu.sync_copy(src_h.at[pl.ds(base, ECH)], sidx)
            pltpu.sync_copy(dst_h.at[pl.ds(base, ECH)], didx)
            pltpu.async_copy(zt.at[sidx], rows, sem).wait()
            pltpu.sync_copy(rows, acc.at[didx], add=True)

        plsc.subcore_barrier()
        pltpu.sync_copy(acc.at[pl.ds(s * NRW, NRW)],
                        esum2p_h.at[c, ci, pl.ds(s * NRW, NRW)])
        plsc.subcore_barrier()


def _sc_pool(h2_h, batch_h, zerosp_h, poolp_h, bidx, rows, acc):
    c = lax.axis_index("c")
    s = lax.axis_index("s")
    w = s * NC + c
    pltpu.sync_copy(zerosp_h.at[pl.ds(s * GRW, GRW)], acc.at[pl.ds(s * GRW, GRW)])
    plsc.subcore_barrier()
    ntrips = jnp.where(w < PNCH - 3 * NW, 4, 3)

    @pl.loop(0, ntrips)
    def _(t):
        base = (w + NW * t) * PCH
        pltpu.sync_copy(batch_h.at[pl.ds(base, PCH)], bidx)
        pltpu.sync_copy(h2_h.at[pl.ds(base, PCH)], rows)
        pltpu.sync_copy(rows, acc.at[bidx], add=True)

    plsc.subcore_barrier()
    pltpu.sync_copy(acc.at[pl.ds(s * GRW, GRW)], poolp_h.at[c, pl.ds(s * GRW, GRW)])


def _tc_prep(xpad_ref, degp_ref, xsp_ref, dinv_ref):
    deg = degp_ref[0, :, 0:1] + degp_ref[1, :, 0:1] + 1.0
    dinv = lax.rsqrt(deg)
    xsp_ref[...] = xpad_ref[...] * dinv
    dinv_ref[...] = jnp.broadcast_to(dinv, (BB, 16))


def _tc_mid(e1_ref, xsp_ref, dinv_ref, w1_ref, b1_ref,
            w2c0, w2c1, w2c2, w2c3, z0_ref, z1_ref, z2_ref, z3_ref):
    agg = (e1_ref[0] + e1_ref[1] + xsp_ref[...]) * dinv_ref[...]
    h1 = jnp.maximum(
        jnp.dot(agg, w1_ref[...], preferred_element_type=f32) + b1_ref[...], 0.0)
    for zref, wc in ((z0_ref, w2c0), (z1_ref, w2c1), (z2_ref, w2c2), (z3_ref, w2c3)):
        zref[...] = jnp.dot(h1, wc[...], preferred_element_type=f32) * dinv_ref[...]


def _tc_h2(e2_ref, z0_ref, z1_ref, z2_ref, z3_ref, dinv_ref, b2_ref, h2_ref):
    parts = []
    for ci, zref in enumerate((z0_ref, z1_ref, z2_ref, z3_ref)):
        es = e2_ref[0, ci] + e2_ref[1, ci]
        parts.append((es + zref[...]) * dinv_ref[...])
    h2_ref[...] = jnp.maximum(
        jnp.concatenate(parts, axis=-1) + b2_ref[...], 0.0)


def _tc_head(poolp_ref, cntp_ref, wf1, bf1, wf2, bf2, wf3, bf3, out_ref):
    cnt = cntp_ref[0, :, 0:1] + cntp_ref[1, :, 0:1]
    pooled = (poolp_ref[0] + poolp_ref[1]) / jnp.maximum(cnt, 1.0)
    a = jnp.maximum(
        jnp.dot(pooled, wf1[...], preferred_element_type=f32) + bf1[...], 0.0)
    a = jnp.maximum(
        jnp.dot(a, wf2[...], preferred_element_type=f32) + bf2[...], 0.0)
    out_ref[...] = jnp.dot(a, wf3[...], preferred_element_type=f32) + bf3[...]


def kernel(x, edge_index, batch, W1, b1, W2, b2, Wf1, bf1, Wf2, bf2, Wf3, bf3):
    src = edge_index[0]
    dst = edge_index[1]
    xpad = jnp.pad(x, ((0, NP - N), (0, 16 - x.shape[1])))
    W1p = jnp.pad(W1, ((0, 16 - W1.shape[0]), (0, 0)))
    w2c = [W2[:, 16 * i:16 * i + 16] for i in range(4)]
    b1r = b1.reshape(1, 128)
    b2r = b2.reshape(1, 64)
    bf1r = bf1.reshape(1, 128)
    bf2r = bf2.reshape(1, 64)
    bf3r = bf3.reshape(1, 1)
    zeros_n = jnp.zeros((NP, 16), f32)
    zeros_p = jnp.zeros((G, 64), f32)
    ones_e = jnp.ones((ECH, 16), f32)

    mesh = plsc.VectorSubcoreMesh(core_axis_name="c", subcore_axis_name="s")
    sc_params = pltpu.CompilerParams(use_tc_tiling_on_sc=False)

    degp, cntp = pl.kernel(
        _sc_deg_cnt,
        out_type=[jax.ShapeDtypeStruct((NC, NP, 16), f32),
                  jax.ShapeDtypeStruct((NC, G, 16), f32)],
        mesh=mesh,
        compiler_params=sc_params,
        scratch_types=[pltpu.VMEM((ECH,), jnp.int32),
                       pltpu.VMEM((ECH, 16), f32),
                       pltpu.VMEM_SHARED((NP, 16), f32),
                       pltpu.VMEM_SHARED((G, 16), f32)],
    )(dst, batch, ones_e, zeros_n)

    xsp, dinvb = pl.pallas_call(
        _tc_prep,
        grid=(NB,),
        in_specs=[pl.BlockSpec((BB, 16), lambda i: (i, 0)),
                  pl.BlockSpec((NC, BB, 16), lambda i: (0, i, 0))],
        out_specs=[pl.BlockSpec((BB, 16), lambda i: (i, 0)),
                   pl.BlockSpec((BB, 16), lambda i: (i, 0))],
        out_shape=[jax.ShapeDtypeStruct((NP, 16), f32),
                   jax.ShapeDtypeStruct((NP, 16), f32)],
    )(xpad, degp)

    esum1p = pl.kernel(
        _sc_agg1,
        out_type=jax.ShapeDtypeStruct((NC, NP, 16), f32),
        mesh=mesh,
        compiler_params=sc_params,
        scratch_types=[pltpu.VMEM((ECH,), jnp.int32),
                       pltpu.VMEM((ECH,), jnp.int32),
                       pltpu.VMEM((ECH, 16), f32),
                       pltpu.VMEM_SHARED((NP, 16), f32),
                       pltpu.SemaphoreType.DMA],
    )(src, dst, xsp, zeros_n)

    z = pl.pallas_call(
        _tc_mid,
        grid=(NB,),
        in_specs=[pl.BlockSpec((NC, BB, 16), lambda i: (0, i, 0)),
                  pl.BlockSpec((BB, 16), lambda i: (i, 0)),
                  pl.BlockSpec((BB, 16), lambda i: (i, 0)),
                  pl.BlockSpec((16, 128), lambda i: (0, 0)),
                  pl.BlockSpec((1, 128), lambda i: (0, 0))] +
                 [pl.BlockSpec((128, 16), lambda i: (0, 0))] * 4,
        out_specs=[pl.BlockSpec((BB, 16), lambda i: (i, 0))] * 4,
        out_shape=[jax.ShapeDtypeStruct((NP, 16), f32)] * 4,
    )(esum1p, xsp, dinvb, W1p, b1r, *w2c)

    esum2p = pl.kernel(
        _sc_agg2,
        out_type=jax.ShapeDtypeStruct((NC, 4, NP, 16), f32),
        mesh=mesh,
        compiler_params=sc_params,
        scratch_types=[pltpu.VMEM((ECH,), jnp.int32),
                       pltpu.VMEM((ECH,), jnp.int32),
                       pltpu.VMEM((ECH, 16), f32),
                       pltpu.VMEM_SHARED((NP, 16), f32),
                       pltpu.SemaphoreType.DMA],
    )(src, dst, *z, zeros_n)

    h2 = pl.pallas_call(
        _tc_h2,
        grid=(NB,),
        in_specs=[pl.BlockSpec((NC, 4, BB, 16), lambda i: (0, 0, i, 0))] +
                 [pl.BlockSpec((BB, 16), lambda i: (i, 0))] * 5 +
                 [pl.BlockSpec((1, 64), lambda i: (0, 0))],
        out_specs=pl.BlockSpec((BB, 64), lambda i: (i, 0)),
        out_shape=jax.ShapeDtypeStruct((NP, 64), f32),
    )(esum2p, *z, dinvb, b2r)

    poolp = pl.kernel(
        _sc_pool,
        out_type=jax.ShapeDtypeStruct((NC, G, 64), f32),
        mesh=mesh,
        compiler_params=sc_params,
        scratch_types=[pltpu.VMEM((PCH,), jnp.int32),
                       pltpu.VMEM((PCH, 64), f32),
                       pltpu.VMEM_SHARED((G, 64), f32)],
    )(h2, batch, zeros_p)

    out = pl.pallas_call(
        _tc_head,
        out_shape=jax.ShapeDtypeStruct((G, 1), f32),
    )(poolp, cntp, Wf1, bf1r, Wf2, bf2r, Wf3, bf3r)

    return out


# speedup vs baseline: 20.4606x; 1.0191x over previous
import jax
import jax.numpy as jnp
from jax import lax
from jax.experimental import pallas as pl
from jax.experimental.pallas import tpu as pltpu
from jax.experimental.pallas import tpu_sc as plsc

f32 = jnp.float32

N = 100000
NP = 100096
E = 1600000
G = 2048
NC = 2
NS = 16
NW = NC * NS
EC = E // NW
ECH = 400
ENCH = EC // ECH
NRW = NP // NS
GRW = G // NS
BCH = N // ECH
PCH = 1000
PNCH = N // PCH

BB = 3128
NB = NP // BB


def _sc_deg_cnt(dst_h, batch_h, ones_h, zeros_h, degp_h, cntp_h,
                idx0, idx1, ones_v, dacc, cacc, sem0, sem1):
    c = lax.axis_index("c")
    s = lax.axis_index("s")
    w = s * NC + c
    pltpu.sync_copy(zeros_h.at[pl.ds(s * NRW, NRW)], dacc.at[pl.ds(s * NRW, NRW)])
    pltpu.sync_copy(zeros_h.at[pl.ds(s * GRW, GRW)], cacc.at[pl.ds(s * GRW, GRW)])
    pltpu.sync_copy(ones_h, ones_v)
    plsc.subcore_barrier()

    def start(k, idx, sem):
        pltpu.async_copy(dst_h.at[pl.ds(w * EC + k * ECH, ECH)], idx, sem)

    def fin(k, idx, sem):
        pltpu.make_async_copy(dst_h.at[pl.ds(w * EC + k * ECH, ECH)], idx, sem).wait()
        pltpu.sync_copy(ones_v, dacc.at[idx], add=True)

    start(0, idx0, sem0)

    @pl.loop(0, (ENCH - 1) // 2)
    def _(j):
        start(2 * j + 1, idx1, sem1)
        fin(2 * j, idx0, sem0)
        start(2 * j + 2, idx0, sem0)
        fin(2 * j + 1, idx1, sem1)

    fin(ENCH - 1, idx0, sem0)

    ncnt = jnp.where(w < BCH - 7 * NW, 8, 7)

    @pl.loop(0, ncnt)
    def _(t):
        pltpu.sync_copy(batch_h.at[pl.ds((w + NW * t) * ECH, ECH)], idx0)
        pltpu.sync_copy(ones_v, cacc.at[idx0], add=True)

    plsc.subcore_barrier()
    pltpu.sync_copy(dacc.at[pl.ds(s * NRW, NRW)], degp_h.at[c, pl.ds(s * NRW, NRW)])
    pltpu.sync_copy(cacc.at[pl.ds(s * GRW, GRW)], cntp_h.at[c, pl.ds(s * GRW, GRW)])


def _agg_pass(table_h, src_h, dst_h, acc, w,
              sidx0, didx0, sidx1, didx1, rows0, rows1, sem0, sem1):
    def start(k, sidx, didx, rows, sem):
        base = w * EC + k * ECH
        pltpu.sync_copy(src_h.at[pl.ds(base, ECH)], sidx)
        pltpu.sync_copy(dst_h.at[pl.ds(base, ECH)], didx)
        pltpu.async_copy(table_h.at[sidx], rows, sem)

    def fin(sidx, didx, rows, sem):
        pltpu.make_async_copy(table_h.at[sidx], rows, sem).wait()
        pltpu.sync_copy(rows, acc.at[didx], add=True)

    start(0, sidx0, didx0, rows0, sem0)

    @pl.loop(0, (ENCH - 1) // 2)
    def _(j):
        start(2 * j + 1, sidx1, didx1, rows1, sem1)
        fin(sidx0, didx0, rows0, sem0)
        start(2 * j + 2, sidx0, didx0, rows0, sem0)
        fin(sidx1, didx1, rows1, sem1)

    fin(sidx0, didx0, rows0, sem0)


def _sc_agg1(src_h, dst_h, xsp_h, zeros_h, esum1p_h,
             sidx0, didx0, sidx1, didx1, rows0, rows1, acc, sem0, sem1):
    c = lax.axis_index("c")
    s = lax.axis_index("s")
    w = s * NC + c
    pltpu.sync_copy(zeros_h.at[pl.ds(s * NRW, NRW)], acc.at[pl.ds(s * NRW, NRW)])
    plsc.subcore_barrier()
    _agg_pass(xsp_h, src_h, dst_h, acc, w,
              sidx0, didx0, sidx1, didx1, rows0, rows1, sem0, sem1)
    plsc.subcore_barrier()
    pltpu.sync_copy(acc.at[pl.ds(s * NRW, NRW)], esum1p_h.at[c, pl.ds(s * NRW, NRW)])


def _sc_agg2(src_h, dst_h, z0, z1, z2, z3, zeros_h, esum2p_h,
             sidx0, didx0, sidx1, didx1, rows0, rows1, acc, sem0, sem1):
    c = lax.axis_index("c")
    s = lax.axis_index("s")
    w = s * NC + c
    for ci, zt in enumerate((z0, z1, z2, z3)):
        pltpu.sync_copy(zeros_h.at[pl.ds(s * NRW, NRW)], acc.at[pl.ds(s * NRW, NRW)])
        plsc.subcore_barrier()
        _agg_pass(zt, src_h, dst_h, acc, w,
                  sidx0, didx0, sidx1, didx1, rows0, rows1, sem0, sem1)
        plsc.subcore_barrier()
        pltpu.sync_copy(acc.at[pl.ds(s * NRW, NRW)],
                        esum2p_h.at[c, ci, pl.ds(s * NRW, NRW)])
        plsc.subcore_barrier()


def _sc_pool(h2_h, batch_h, zerosp_h, poolp_h, bidx, rows, acc):
    c = lax.axis_index("c")
    s = lax.axis_index("s")
    w = s * NC + c
    pltpu.sync_copy(zerosp_h.at[pl.ds(s * GRW, GRW)], acc.at[pl.ds(s * GRW, GRW)])
    plsc.subcore_barrier()
    ntrips = jnp.where(w < PNCH - 3 * NW, 4, 3)

    @pl.loop(0, ntrips)
    def _(t):
        base = (w + NW * t) * PCH
        pltpu.sync_copy(batch_h.at[pl.ds(base, PCH)], bidx)
        pltpu.sync_copy(h2_h.at[pl.ds(base, PCH)], rows)
        pltpu.sync_copy(rows, acc.at[bidx], add=True)

    plsc.subcore_barrier()
    pltpu.sync_copy(acc.at[pl.ds(s * GRW, GRW)], poolp_h.at[c, pl.ds(s * GRW, GRW)])


def _tc_prep(xpad_ref, degp_ref, xsp_ref, dinv_ref):
    deg = degp_ref[0, :, 0:1] + degp_ref[1, :, 0:1] + 1.0
    dinv = lax.rsqrt(deg)
    xsp_ref[...] = xpad_ref[...] * dinv
    dinv_ref[...] = jnp.broadcast_to(dinv, (BB, 16))


def _tc_mid(e1_ref, xsp_ref, dinv_ref, w1_ref, b1_ref,
            w2c0, w2c1, w2c2, w2c3, z0_ref, z1_ref, z2_ref, z3_ref):
    agg = (e1_ref[0] + e1_ref[1] + xsp_ref[...]) * dinv_ref[...]
    h1 = jnp.maximum(
        jnp.dot(agg, w1_ref[...], preferred_element_type=f32) + b1_ref[...], 0.0)
    for zref, wc in ((z0_ref, w2c0), (z1_ref, w2c1), (z2_ref, w2c2), (z3_ref, w2c3)):
        zref[...] = jnp.dot(h1, wc[...], preferred_element_type=f32) * dinv_ref[...]


def _tc_h2(e2_ref, z0_ref, z1_ref, z2_ref, z3_ref, dinv_ref, b2_ref, h2_ref):
    parts = []
    for ci, zref in enumerate((z0_ref, z1_ref, z2_ref, z3_ref)):
        es = e2_ref[0, ci] + e2_ref[1, ci]
        parts.append((es + zref[...]) * dinv_ref[...])
    h2_ref[...] = jnp.maximum(
        jnp.concatenate(parts, axis=-1) + b2_ref[...], 0.0)


def _tc_head(poolp_ref, cntp_ref, wf1, bf1, wf2, bf2, wf3, bf3, out_ref):
    cnt = cntp_ref[0, :, 0:1] + cntp_ref[1, :, 0:1]
    pooled = (poolp_ref[0] + poolp_ref[1]) / jnp.maximum(cnt, 1.0)
    a = jnp.maximum(
        jnp.dot(pooled, wf1[...], preferred_element_type=f32) + bf1[...], 0.0)
    a = jnp.maximum(
        jnp.dot(a, wf2[...], preferred_element_type=f32) + bf2[...], 0.0)
    out_ref[...] = jnp.dot(a, wf3[...], preferred_element_type=f32) + bf3[...]


def kernel(x, edge_index, batch, W1, b1, W2, b2, Wf1, bf1, Wf2, bf2, Wf3, bf3):
    src = edge_index[0]
    dst = edge_index[1]
    xpad = jnp.pad(x, ((0, NP - N), (0, 16 - x.shape[1])))
    W1p = jnp.pad(W1, ((0, 16 - W1.shape[0]), (0, 0)))
    w2c = [W2[:, 16 * i:16 * i + 16] for i in range(4)]
    b1r = b1.reshape(1, 128)
    b2r = b2.reshape(1, 64)
    bf1r = bf1.reshape(1, 128)
    bf2r = bf2.reshape(1, 64)
    bf3r = bf3.reshape(1, 1)
    zeros_n = jnp.zeros((NP, 16), f32)
    zeros_p = jnp.zeros((G, 64), f32)
    ones_e = jnp.ones((ECH, 16), f32)

    mesh = plsc.VectorSubcoreMesh(core_axis_name="c", subcore_axis_name="s")
    sc_params = pltpu.CompilerParams(use_tc_tiling_on_sc=False)

    degp, cntp = pl.kernel(
        _sc_deg_cnt,
        out_type=[jax.ShapeDtypeStruct((NC, NP, 16), f32),
                  jax.ShapeDtypeStruct((NC, G, 16), f32)],
        mesh=mesh,
        compiler_params=sc_params,
        scratch_types=[pltpu.VMEM((ECH,), jnp.int32),
                       pltpu.VMEM((ECH,), jnp.int32),
                       pltpu.VMEM((ECH, 16), f32),
                       pltpu.VMEM_SHARED((NP, 16), f32),
                       pltpu.VMEM_SHARED((G, 16), f32),
                       pltpu.SemaphoreType.DMA,
                       pltpu.SemaphoreType.DMA],
    )(dst, batch, ones_e, zeros_n)

    xsp, dinvb = pl.pallas_call(
        _tc_prep,
        grid=(NB,),
        in_specs=[pl.BlockSpec((BB, 16), lambda i: (i, 0)),
                  pl.BlockSpec((NC, BB, 16), lambda i: (0, i, 0))],
        out_specs=[pl.BlockSpec((BB, 16), lambda i: (i, 0)),
                   pl.BlockSpec((BB, 16), lambda i: (i, 0))],
        out_shape=[jax.ShapeDtypeStruct((NP, 16), f32),
                   jax.ShapeDtypeStruct((NP, 16), f32)],
    )(xpad, degp)

    agg_scratch = [pltpu.VMEM((ECH,), jnp.int32),
                   pltpu.VMEM((ECH,), jnp.int32),
                   pltpu.VMEM((ECH,), jnp.int32),
                   pltpu.VMEM((ECH,), jnp.int32),
                   pltpu.VMEM((ECH, 16), f32),
                   pltpu.VMEM((ECH, 16), f32),
                   pltpu.VMEM_SHARED((NP, 16), f32),
                   pltpu.SemaphoreType.DMA,
                   pltpu.SemaphoreType.DMA]

    esum1p = pl.kernel(
        _sc_agg1,
        out_type=jax.ShapeDtypeStruct((NC, NP, 16), f32),
        mesh=mesh,
        compiler_params=sc_params,
        scratch_types=agg_scratch,
    )(src, dst, xsp, zeros_n)

    z = pl.pallas_call(
        _tc_mid,
        grid=(NB,),
        in_specs=[pl.BlockSpec((NC, BB, 16), lambda i: (0, i, 0)),
                  pl.BlockSpec((BB, 16), lambda i: (i, 0)),
                  pl.BlockSpec((BB, 16), lambda i: (i, 0)),
                  pl.BlockSpec((16, 128), lambda i: (0, 0)),
                  pl.BlockSpec((1, 128), lambda i: (0, 0))] +
                 [pl.BlockSpec((128, 16), lambda i: (0, 0))] * 4,
        out_specs=[pl.BlockSpec((BB, 16), lambda i: (i, 0))] * 4,
        out_shape=[jax.ShapeDtypeStruct((NP, 16), f32)] * 4,
    )(esum1p, xsp, dinvb, W1p, b1r, *w2c)

    esum2p = pl.kernel(
        _sc_agg2,
        out_type=jax.ShapeDtypeStruct((NC, 4, NP, 16), f32),
        mesh=mesh,
        compiler_params=sc_params,
        scratch_types=agg_scratch,
    )(src, dst, *z, zeros_n)

    h2 = pl.pallas_call(
        _tc_h2,
        grid=(NB,),
        in_specs=[pl.BlockSpec((NC, 4, BB, 16), lambda i: (0, 0, i, 0))] +
                 [pl.BlockSpec((BB, 16), lambda i: (i, 0))] * 5 +
                 [pl.BlockSpec((1, 64), lambda i: (0, 0))],
        out_specs=pl.BlockSpec((BB, 64), lambda i: (i, 0)),
        out_shape=jax.ShapeDtypeStruct((NP, 64), f32),
    )(esum2p, *z, dinvb, b2r)

    poolp = pl.kernel(
        _sc_pool,
        out_type=jax.ShapeDtypeStruct((NC, G, 64), f32),
        mesh=mesh,
        compiler_params=sc_params,
        scratch_types=[pltpu.VMEM((PCH,), jnp.int32),
                       pltpu.VMEM((PCH, 64), f32),
                       pltpu.VMEM_SHARED((G, 64), f32)],
    )(h2, batch, zeros_p)

    out = pl.pallas_call(
        _tc_head,
        out_shape=jax.ShapeDtypeStruct((G, 1), f32),
    )(poolp, cntp, Wf1, bf1r, Wf2, bf2r, Wf3, bf3r)

    return out


# speedup vs baseline: 23.3118x; 1.1394x over previous
import jax
import jax.numpy as jnp
from jax import lax
from jax.experimental import pallas as pl
from jax.experimental.pallas import tpu as pltpu
from jax.experimental.pallas import tpu_sc as plsc

f32 = jnp.float32

N = 100000
NP = 100096
E = 1600000
G = 2048
GP = 2176
NC = 2
NS = 16
NW = NC * NS
EC = E // NW
EC2 = E // NS
ECH = 400
ENCH = EC // ECH
ENCH2 = EC2 // ECH
NRW = NP // NS
GRW = G // NS
H2T = NRW - 15 * ECH

BB = 3128
NB = NP // BB


def _sc_deg_cnt(dst_h, batch_h, ones_h, zeros_h, degp_h, cntp_h,
                idx0, idx1, ones_v, dacc, cacc, sem0, sem1):
    c = lax.axis_index("c")
    s = lax.axis_index("s")
    w = s * NC + c
    pltpu.sync_copy(zeros_h.at[pl.ds(s * NRW, NRW)], dacc.at[pl.ds(s * NRW, NRW)])
    pltpu.sync_copy(zeros_h.at[pl.ds(s * GRW, GRW)], cacc.at[pl.ds(s * GRW, GRW)])
    pltpu.sync_copy(ones_h, ones_v)
    plsc.subcore_barrier()

    def start(k, idx, sem):
        pltpu.async_copy(dst_h.at[pl.ds(w * EC + k * ECH, ECH)], idx, sem)

    def fin(k, idx, sem):
        pltpu.make_async_copy(dst_h.at[pl.ds(w * EC + k * ECH, ECH)], idx, sem).wait()
        pltpu.sync_copy(ones_v, dacc.at[idx], add=True)

    start(0, idx0, sem0)

    @pl.loop(0, (ENCH - 1) // 2)
    def _(j):
        start(2 * j + 1, idx1, sem1)
        fin(2 * j, idx0, sem0)
        start(2 * j + 2, idx0, sem0)
        fin(2 * j + 1, idx1, sem1)

    fin(ENCH - 1, idx0, sem0)

    ncnt = jnp.where(w < (N // ECH) - 7 * NW, 8, 7)

    @pl.loop(0, ncnt)
    def _(t):
        pltpu.sync_copy(batch_h.at[pl.ds((w + NW * t) * ECH, ECH)], idx0)
        pltpu.sync_copy(ones_v, cacc.at[idx0], add=True)

    plsc.subcore_barrier()
    pltpu.sync_copy(dacc.at[pl.ds(s * NRW, NRW)], degp_h.at[c, pl.ds(s * NRW, NRW)])
    pltpu.sync_copy(cacc.at[pl.ds(s * GRW, GRW)], cntp_h.at[c, pl.ds(s * GRW, GRW)])


def _agg_pass(table_r, src_h, dst_h, acc, base0, nch,
              sidx0, didx0, sidx1, didx1, rows0, rows1, sem0, sem1):
    def start(k, sidx, didx, rows, sem):
        base = base0 + k * ECH
        pltpu.sync_copy(src_h.at[pl.ds(base, ECH)], sidx)
        pltpu.sync_copy(dst_h.at[pl.ds(base, ECH)], didx)
        pltpu.async_copy(table_r.at[sidx], rows, sem)

    def fin(sidx, didx, rows, sem):
        pltpu.make_async_copy(table_r.at[sidx], rows, sem).wait()
        pltpu.sync_copy(rows, acc.at[didx], add=True)

    start(0, sidx0, didx0, rows0, sem0)
    npairs = (nch - 1) // 2 if nch % 2 else (nch - 2) // 2

    @pl.loop(0, npairs)
    def _(j):
        start(2 * j + 1, sidx1, didx1, rows1, sem1)
        fin(sidx0, didx0, rows0, sem0)
        start(2 * j + 2, sidx0, didx0, rows0, sem0)
        fin(sidx1, didx1, rows1, sem1)

    if nch % 2:
        fin(sidx0, didx0, rows0, sem0)
    else:
        start(nch - 1, sidx1, didx1, rows1, sem1)
        fin(sidx0, didx0, rows0, sem0)
        fin(sidx1, didx1, rows1, sem1)


def _sc_agg1(src_h, dst_h, xsp_h, zeros_h, esum1p_h,
             sidx0, didx0, sidx1, didx1, rows0, rows1, acc, sem0, sem1):
    c = lax.axis_index("c")
    s = lax.axis_index("s")
    w = s * NC + c
    pltpu.sync_copy(zeros_h.at[pl.ds(s * NRW, NRW)], acc.at[pl.ds(s * NRW, NRW)])
    plsc.subcore_barrier()
    _agg_pass(xsp_h, src_h, dst_h, acc, w * EC, ENCH,
              sidx0, didx0, sidx1, didx1, rows0, rows1, sem0, sem1)
    plsc.subcore_barrier()
    pltpu.sync_copy(acc.at[pl.ds(s * NRW, NRW)], esum1p_h.at[c, pl.ds(s * NRW, NRW)])


def _sc_agg2pool(src_h, dst_h, zt_h, dinv_h, batch_h, b2_h, zeros_h, poolp_h,
                 sidx0, didx0, sidx1, didx1, rows0, rows1, dbuf, bidx, bidxt,
                 b2v, acc, pacc0, pacc1, sem0, sem1):
    c = lax.axis_index("c")
    s = lax.axis_index("s")
    pltpu.sync_copy(b2_h, b2v)
    for k, pacc in enumerate((pacc0, pacc1)):
        cidx = 2 * c + k
        table = zt_h.at[cidx]
        pltpu.sync_copy(zeros_h.at[pl.ds(s * NRW, NRW)], acc.at[pl.ds(s * NRW, NRW)])
        pltpu.sync_copy(zeros_h.at[pl.ds(s * (GP // NS), GP // NS)],
                        pacc.at[pl.ds(s * (GP // NS), GP // NS)])
        plsc.subcore_barrier()
        _agg_pass(table, src_h, dst_h, acc, s * EC2, ENCH2,
                  sidx0, didx0, sidx1, didx1, rows0, rows1, sem0, sem1)
        plsc.subcore_barrier()

        b2sel = jnp.where(c == 0, b2v[k], b2v[2 + k])

        def h2_chunk(base, size, brf):
            pltpu.sync_copy(acc.at[pl.ds(base, size)], rows0.at[pl.ds(0, size)])
            pltpu.sync_copy(table.at[pl.ds(base, size)], rows1.at[pl.ds(0, size)])
            pltpu.sync_copy(dinv_h.at[pl.ds(base, size)], dbuf.at[pl.ds(0, size)])
            pltpu.sync_copy(batch_h.at[pl.ds(base, size)], brf)

            @pl.loop(0, size)
            def _(i):
                v = (rows0[i, :] + rows1[i, :]) * dbuf[i, :] + b2sel
                rows0[i, :] = jnp.maximum(v, 0.0)

            pltpu.sync_copy(rows0.at[pl.ds(0, size)], pacc.at[brf], add=True)

        for j in range(15):
            h2_chunk(s * NRW + j * ECH, ECH, bidx)
        h2_chunk(s * NRW + 15 * ECH, H2T, bidxt)
        plsc.subcore_barrier()
        pltpu.sync_copy(pacc.at[pl.ds(s * GRW, GRW)],
                        poolp_h.at[c, k, pl.ds(s * GRW, GRW)])


def _tc_prep(xpad_ref, degp_ref, xsp_ref, dinv_ref):
    deg = degp_ref[0, :, 0:1] + degp_ref[1, :, 0:1] + 1.0
    dinv = lax.rsqrt(deg)
    xsp_ref[...] = xpad_ref[...] * dinv
    dinv_ref[...] = jnp.broadcast_to(dinv, (BB, 16))


def _tc_mid(e1_ref, xsp_ref, dinv_ref, w1_ref, b1_ref, w2_ref, zt_ref):
    agg = (e1_ref[0] + e1_ref[1] + xsp_ref[...]) * dinv_ref[...]
    h1 = jnp.maximum(
        jnp.dot(agg, w1_ref[...], preferred_element_type=f32) + b1_ref[...], 0.0)
    z = jnp.dot(h1, w2_ref[0], preferred_element_type=f32) * dinv_ref[...]
    zt_ref[...] = z[None]


def _tc_head(poolp_ref, cntp_ref, wf1c0, wf1c1, wf1c2, wf1c3,
             bf1, wf2, bf2, wf3, bf3, out_ref):
    cnt = cntp_ref[0, :, 0:1] + cntp_ref[1, :, 0:1]
    recip = 1.0 / jnp.maximum(cnt, 1.0)
    acc = bf1[...]
    for ci, wc in enumerate((wf1c0, wf1c1, wf1c2, wf1c3)):
        pooled = poolp_ref[ci // 2, ci % 2] * recip
        acc = acc + jnp.dot(pooled, wc[...], preferred_element_type=f32)
    a = jnp.maximum(acc, 0.0)
    a = jnp.maximum(
        jnp.dot(a, wf2[...], preferred_element_type=f32) + bf2[...], 0.0)
    out_ref[...] = jnp.dot(a, wf3[...], preferred_element_type=f32) + bf3[...]


def kernel(x, edge_index, batch, W1, b1, W2, b2, Wf1, bf1, Wf2, bf2, Wf3, bf3):
    src = edge_index[0]
    dst = edge_index[1]
    batchp = jnp.pad(batch, (0, NP - N), constant_values=G)
    xpad = jnp.pad(x, ((0, NP - N), (0, 16 - x.shape[1])))
    W1p = jnp.pad(W1, ((0, 16 - W1.shape[0]), (0, 0)))
    w2s = jnp.stack([W2[:, 16 * i:16 * i + 16] for i in range(4)])
    wf1c = [Wf1[16 * i:16 * i + 16, :] for i in range(4)]
    b1r = b1.reshape(1, 128)
    b2q = b2.reshape(4, 16)
    bf1r = bf1.reshape(1, 128)
    bf2r = bf2.reshape(1, 64)
    bf3r = bf3.reshape(1, 1)
    zeros_n = jnp.zeros((NP, 16), f32)
    ones_e = jnp.ones((ECH, 16), f32)

    mesh = plsc.VectorSubcoreMesh(core_axis_name="c", subcore_axis_name="s")
    sc_params = pltpu.CompilerParams(use_tc_tiling_on_sc=False)

    degp, cntp = pl.kernel(
        _sc_deg_cnt,
        out_type=[jax.ShapeDtypeStruct((NC, NP, 16), f32),
                  jax.ShapeDtypeStruct((NC, G, 16), f32)],
        mesh=mesh,
        compiler_params=sc_params,
        scratch_types=[pltpu.VMEM((ECH,), jnp.int32),
                       pltpu.VMEM((ECH,), jnp.int32),
                       pltpu.VMEM((ECH, 16), f32),
                       pltpu.VMEM_SHARED((NP, 16), f32),
                       pltpu.VMEM_SHARED((G, 16), f32),
                       pltpu.SemaphoreType.DMA,
                       pltpu.SemaphoreType.DMA],
    )(dst, batchp, ones_e, zeros_n)

    xsp, dinvb = pl.pallas_call(
        _tc_prep,
        grid=(NB,),
        in_specs=[pl.BlockSpec((BB, 16), lambda i: (i, 0)),
                  pl.BlockSpec((NC, BB, 16), lambda i: (0, i, 0))],
        out_specs=[pl.BlockSpec((BB, 16), lambda i: (i, 0)),
                   pl.BlockSpec((BB, 16), lambda i: (i, 0))],
        out_shape=[jax.ShapeDtypeStruct((NP, 16), f32),
                   jax.ShapeDtypeStruct((NP, 16), f32)],
    )(xpad, degp)

    esum1p = pl.kernel(
        _sc_agg1,
        out_type=jax.ShapeDtypeStruct((NC, NP, 16), f32),
        mesh=mesh,
        compiler_params=sc_params,
        scratch_types=[pltpu.VMEM((ECH,), jnp.int32),
                       pltpu.VMEM((ECH,), jnp.int32),
                       pltpu.VMEM((ECH,), jnp.int32),
                       pltpu.VMEM((ECH,), jnp.int32),
                       pltpu.VMEM((ECH, 16), f32),
                       pltpu.VMEM((ECH, 16), f32),
                       pltpu.VMEM_SHARED((NP, 16), f32),
                       pltpu.SemaphoreType.DMA,
                       pltpu.SemaphoreType.DMA],
    )(src, dst, xsp, zeros_n)

    zt = pl.pallas_call(
        _tc_mid,
        grid=(NB, 4),
        in_specs=[pl.BlockSpec((NC, BB, 16), lambda i, ci: (0, i, 0)),
                  pl.BlockSpec((BB, 16), lambda i, ci: (i, 0)),
                  pl.BlockSpec((BB, 16), lambda i, ci: (i, 0)),
                  pl.BlockSpec((16, 128), lambda i, ci: (0, 0)),
                  pl.BlockSpec((1, 128), lambda i, ci: (0, 0)),
                  pl.BlockSpec((1, 128, 16), lambda i, ci: (ci, 0, 0))],
        out_specs=pl.BlockSpec((1, BB, 16), lambda i, ci: (ci, i, 0)),
        out_shape=jax.ShapeDtypeStruct((4, NP, 16), f32),
    )(esum1p, xsp, dinvb, W1p, b1r, w2s)

    poolp = pl.kernel(
        _sc_agg2pool,
        out_type=jax.ShapeDtypeStruct((NC, 2, G, 16), f32),
        mesh=mesh,
        compiler_params=sc_params,
        scratch_types=[pltpu.VMEM((ECH,), jnp.int32),
                       pltpu.VMEM((ECH,), jnp.int32),
                       pltpu.VMEM((ECH,), jnp.int32),
                       pltpu.VMEM((ECH,), jnp.int32),
                       pltpu.VMEM((ECH, 16), f32),
                       pltpu.VMEM((ECH, 16), f32),
                       pltpu.VMEM((ECH, 16), f32),
                       pltpu.VMEM((ECH,), jnp.int32),
                       pltpu.VMEM((H2T,), jnp.int32),
                       pltpu.VMEM((4, 16), f32),
                       pltpu.VMEM_SHARED((NP, 16), f32),
                       pltpu.VMEM_SHARED((GP, 16), f32),
                       pltpu.VMEM_SHARED((GP, 16), f32),
                       pltpu.SemaphoreType.DMA,
                       pltpu.SemaphoreType.DMA],
    )(src, dst, zt, dinvb, batchp, b2q, zeros_n)

    out = pl.pallas_call(
        _tc_head,
        out_shape=jax.ShapeDtypeStruct((G, 1), f32),
    )(poolp, cntp, *wf1c, bf1r, Wf2, bf2r, Wf3, bf3r)

    return out


# speedup vs baseline: 25.2479x; 1.0831x over previous
import jax
import jax.numpy as jnp
from jax import lax
from jax.experimental import pallas as pl
from jax.experimental.pallas import tpu as pltpu
from jax.experimental.pallas import tpu_sc as plsc

f32 = jnp.float32

N = 100000
NP = 100096
E = 1600000
G = 2048
GP = 2176
NC = 2
NS = 16
NW = NC * NS
EC = E // NW
EC2 = E // NS
ECH = 400
ENCH = EC // ECH
ENCH2 = EC2 // ECH
NRW = NP // NS
GRW = G // NS
H2T = NRW - 15 * ECH

BB = 3128
NB = NP // BB


def _sc_deg_cnt(ei_h, batch_h, ones_h, zeros_h, degp_h, cntp_h,
                idx0, idx1, ones_v, dacc, cacc, sem0, sem1):
    c = lax.axis_index("c")
    s = lax.axis_index("s")
    w = s * NC + c
    pltpu.sync_copy(zeros_h.at[pl.ds(s * NRW, NRW)], dacc.at[pl.ds(s * NRW, NRW)])
    pltpu.sync_copy(zeros_h.at[pl.ds(s * GRW, GRW)], cacc.at[pl.ds(s * GRW, GRW)])
    pltpu.sync_copy(ones_h, ones_v)
    plsc.subcore_barrier()

    def start(k, idx, sem):
        pltpu.async_copy(ei_h.at[1, pl.ds(w * EC + k * ECH, ECH)], idx, sem)

    def fin(k, idx, sem):
        pltpu.make_async_copy(ei_h.at[1, pl.ds(w * EC + k * ECH, ECH)], idx, sem).wait()
        pltpu.sync_copy(ones_v, dacc.at[idx], add=True)

    start(0, idx0, sem0)

    @pl.loop(0, (ENCH - 1) // 2)
    def _(j):
        start(2 * j + 1, idx1, sem1)
        fin(2 * j, idx0, sem0)
        start(2 * j + 2, idx0, sem0)
        fin(2 * j + 1, idx1, sem1)

    fin(ENCH - 1, idx0, sem0)

    ncnt = jnp.where(w < (N // ECH) - 7 * NW, 8, 7)

    @pl.loop(0, ncnt)
    def _(t):
        pltpu.sync_copy(batch_h.at[pl.ds((w + NW * t) * ECH, ECH)], idx0)
        pltpu.sync_copy(ones_v, cacc.at[idx0], add=True)

    plsc.subcore_barrier()
    pltpu.sync_copy(dacc.at[pl.ds(s * NRW, NRW)], degp_h.at[c, pl.ds(s * NRW, NRW)])
    pltpu.sync_copy(cacc.at[pl.ds(s * GRW, GRW)], cntp_h.at[c, pl.ds(s * GRW, GRW)])


def _agg_pass(table_r, ei_h, acc, base0, nch,
              sidx0, didx0, sidx1, didx1, rows0, rows1, sem0, sem1):
    def start(k, sidx, didx, rows, sem):
        base = base0 + k * ECH
        pltpu.sync_copy(ei_h.at[0, pl.ds(base, ECH)], sidx)
        pltpu.sync_copy(ei_h.at[1, pl.ds(base, ECH)], didx)
        pltpu.async_copy(table_r.at[sidx], rows, sem)

    def fin(sidx, didx, rows, sem):
        pltpu.make_async_copy(table_r.at[sidx], rows, sem).wait()
        pltpu.sync_copy(rows, acc.at[didx], add=True)

    start(0, sidx0, didx0, rows0, sem0)
    npairs = (nch - 1) // 2 if nch % 2 else (nch - 2) // 2

    @pl.loop(0, npairs)
    def _(j):
        start(2 * j + 1, sidx1, didx1, rows1, sem1)
        fin(sidx0, didx0, rows0, sem0)
        start(2 * j + 2, sidx0, didx0, rows0, sem0)
        fin(sidx1, didx1, rows1, sem1)

    if nch % 2:
        fin(sidx0, didx0, rows0, sem0)
    else:
        start(nch - 1, sidx1, didx1, rows1, sem1)
        fin(sidx0, didx0, rows0, sem0)
        fin(sidx1, didx1, rows1, sem1)


def _sc_agg1(ei_h, xsp_h, zeros_h, esum1p_h,
             sidx0, didx0, sidx1, didx1, rows0, rows1, acc, sem0, sem1):
    c = lax.axis_index("c")
    s = lax.axis_index("s")
    w = s * NC + c
    pltpu.sync_copy(zeros_h.at[pl.ds(s * NRW, NRW)], acc.at[pl.ds(s * NRW, NRW)])
    plsc.subcore_barrier()
    _agg_pass(xsp_h, ei_h, acc, w * EC, ENCH,
              sidx0, didx0, sidx1, didx1, rows0, rows1, sem0, sem1)
    plsc.subcore_barrier()
    pltpu.sync_copy(acc.at[pl.ds(s * NRW, NRW)], esum1p_h.at[c, pl.ds(s * NRW, NRW)])


def _sc_agg2pool(ei_h, zt_h, dinv_h, batch_h, b2_h, zeros_h, poolp_h,
                 sidx0, didx0, sidx1, didx1, rows0, rows1, dbuf, bidx, bidxt,
                 b2v, acc, pacc0, pacc1, sem0, sem1):
    c = lax.axis_index("c")
    s = lax.axis_index("s")
    pltpu.sync_copy(b2_h, b2v)
    for k, pacc in enumerate((pacc0, pacc1)):
        cidx = 2 * c + k
        table = zt_h.at[cidx]
        pltpu.sync_copy(zeros_h.at[pl.ds(s * NRW, NRW)], acc.at[pl.ds(s * NRW, NRW)])
        pltpu.sync_copy(zeros_h.at[pl.ds(s * (GP // NS), GP // NS)],
                        pacc.at[pl.ds(s * (GP // NS), GP // NS)])
        plsc.subcore_barrier()
        _agg_pass(table, ei_h, acc, s * EC2, ENCH2,
                  sidx0, didx0, sidx1, didx1, rows0, rows1, sem0, sem1)
        plsc.subcore_barrier()

        b2sel = jnp.where(c == 0, b2v[k], b2v[2 + k])

        def h2_chunk(base, size, brf):
            pltpu.sync_copy(acc.at[pl.ds(base, size)], rows0.at[pl.ds(0, size)])
            pltpu.sync_copy(table.at[pl.ds(base, size)], rows1.at[pl.ds(0, size)])
            pltpu.sync_copy(dinv_h.at[pl.ds(base, size)], dbuf.at[pl.ds(0, size)])
            pltpu.sync_copy(batch_h.at[pl.ds(base, size)], brf)

            @pl.loop(0, size)
            def _(i):
                v = (rows0[i, :] + rows1[i, :]) * dbuf[i, :] + b2sel
                rows0[i, :] = jnp.maximum(v, 0.0)

            pltpu.sync_copy(rows0.at[pl.ds(0, size)], pacc.at[brf], add=True)

        for j in range(15):
            h2_chunk(s * NRW + j * ECH, ECH, bidx)
        h2_chunk(s * NRW + 15 * ECH, H2T, bidxt)
        plsc.subcore_barrier()
        pltpu.sync_copy(pacc.at[pl.ds(s * GRW, GRW)],
                        poolp_h.at[c, k, pl.ds(s * GRW, GRW)])


def _tc_prep(xpad_ref, degp_ref, xsp_ref, dinv_ref):
    deg = degp_ref[0, :, 0:1] + degp_ref[1, :, 0:1] + 1.0
    dinv = lax.rsqrt(deg)
    xsp_ref[...] = xpad_ref[...] * dinv
    dinv_ref[...] = jnp.broadcast_to(dinv, (BB, 16))


def _tc_mid(e1_ref, xsp_ref, dinv_ref, w1_ref, b1_ref, w2_ref, zt_ref):
    agg = (e1_ref[0] + e1_ref[1] + xsp_ref[...]) * dinv_ref[...]
    h1 = jnp.maximum(
        jnp.dot(agg, w1_ref[...], preferred_element_type=f32) + b1_ref[...], 0.0)
    zs = [jnp.dot(h1, w2_ref[ci], preferred_element_type=f32) * dinv_ref[...]
          for ci in range(4)]
    zt_ref[...] = jnp.stack(zs, axis=0)


def _tc_head(poolp_ref, cntp_ref, wf1c0, wf1c1, wf1c2, wf1c3,
             bf1, wf2, bf2, wf3, bf3, out_ref):
    cnt = cntp_ref[0, :, 0:1] + cntp_ref[1, :, 0:1]
    recip = 1.0 / jnp.maximum(cnt, 1.0)
    acc = bf1[...]
    for ci, wc in enumerate((wf1c0, wf1c1, wf1c2, wf1c3)):
        pooled = poolp_ref[ci // 2, ci % 2] * recip
        acc = acc + jnp.dot(pooled, wc[...], preferred_element_type=f32)
    a = jnp.maximum(acc, 0.0)
    a = jnp.maximum(
        jnp.dot(a, wf2[...], preferred_element_type=f32) + bf2[...], 0.0)
    out_ref[...] = jnp.dot(a, wf3[...], preferred_element_type=f32) + bf3[...]


def kernel(x, edge_index, batch, W1, b1, W2, b2, Wf1, bf1, Wf2, bf2, Wf3, bf3):
    batchp = jnp.pad(batch, (0, NP - N), constant_values=G)
    xpad = jnp.pad(x, ((0, NP - N), (0, 16 - x.shape[1])))
    W1p = jnp.pad(W1, ((0, 16 - W1.shape[0]), (0, 0)))
    w2s = jnp.stack([W2[:, 16 * i:16 * i + 16] for i in range(4)])
    wf1c = [Wf1[16 * i:16 * i + 16, :] for i in range(4)]
    b1r = b1.reshape(1, 128)
    b2q = b2.reshape(4, 16)
    bf1r = bf1.reshape(1, 128)
    bf2r = bf2.reshape(1, 64)
    bf3r = bf3.reshape(1, 1)
    zeros_n = jnp.zeros((NP, 16), f32)
    ones_e = jnp.ones((ECH, 16), f32)

    mesh = plsc.VectorSubcoreMesh(core_axis_name="c", subcore_axis_name="s")
    sc_params = pltpu.CompilerParams(use_tc_tiling_on_sc=False)

    degp, cntp = pl.kernel(
        _sc_deg_cnt,
        out_type=[jax.ShapeDtypeStruct((NC, NP, 16), f32),
                  jax.ShapeDtypeStruct((NC, G, 16), f32)],
        mesh=mesh,
        compiler_params=sc_params,
        scratch_types=[pltpu.VMEM((ECH,), jnp.int32),
                       pltpu.VMEM((ECH,), jnp.int32),
                       pltpu.VMEM((ECH, 16), f32),
                       pltpu.VMEM_SHARED((NP, 16), f32),
                       pltpu.VMEM_SHARED((G, 16), f32),
                       pltpu.SemaphoreType.DMA,
                       pltpu.SemaphoreType.DMA],
    )(edge_index, batchp, ones_e, zeros_n)

    xsp, dinvb = pl.pallas_call(
        _tc_prep,
        grid=(NB,),
        in_specs=[pl.BlockSpec((BB, 16), lambda i: (i, 0)),
                  pl.BlockSpec((NC, BB, 16), lambda i: (0, i, 0))],
        out_specs=[pl.BlockSpec((BB, 16), lambda i: (i, 0)),
                   pl.BlockSpec((BB, 16), lambda i: (i, 0))],
        out_shape=[jax.ShapeDtypeStruct((NP, 16), f32),
                   jax.ShapeDtypeStruct((NP, 16), f32)],
    )(xpad, degp)

    esum1p = pl.kernel(
        _sc_agg1,
        out_type=jax.ShapeDtypeStruct((NC, NP, 16), f32),
        mesh=mesh,
        compiler_params=sc_params,
        scratch_types=[pltpu.VMEM((ECH,), jnp.int32),
                       pltpu.VMEM((ECH,), jnp.int32),
                       pltpu.VMEM((ECH,), jnp.int32),
                       pltpu.VMEM((ECH,), jnp.int32),
                       pltpu.VMEM((ECH, 16), f32),
                       pltpu.VMEM((ECH, 16), f32),
                       pltpu.VMEM_SHARED((NP, 16), f32),
                       pltpu.SemaphoreType.DMA,
                       pltpu.SemaphoreType.DMA],
    )(edge_index, xsp, zeros_n)

    zt = pl.pallas_call(
        _tc_mid,
        grid=(NB,),
        in_specs=[pl.BlockSpec((NC, BB, 16), lambda i: (0, i, 0)),
                  pl.BlockSpec((BB, 16), lambda i: (i, 0)),
                  pl.BlockSpec((BB, 16), lambda i: (i, 0)),
                  pl.BlockSpec((16, 128), lambda i: (0, 0)),
                  pl.BlockSpec((1, 128), lambda i: (0, 0)),
                  pl.BlockSpec((4, 128, 16), lambda i: (0, 0, 0))],
        out_specs=pl.BlockSpec((4, BB, 16), lambda i: (0, i, 0)),
        out_shape=jax.ShapeDtypeStruct((4, NP, 16), f32),
    )(esum1p, xsp, dinvb, W1p, b1r, w2s)

    poolp = pl.kernel(
        _sc_agg2pool,
        out_type=jax.ShapeDtypeStruct((NC, 2, G, 16), f32),
        mesh=mesh,
        compiler_params=sc_params,
        scratch_types=[pltpu.VMEM((ECH,), jnp.int32),
                       pltpu.VMEM((ECH,), jnp.int32),
                       pltpu.VMEM((ECH,), jnp.int32),
                       pltpu.VMEM((ECH,), jnp.int32),
                       pltpu.VMEM((ECH, 16), f32),
                       pltpu.VMEM((ECH, 16), f32),
                       pltpu.VMEM((ECH, 16), f32),
                       pltpu.VMEM((ECH,), jnp.int32),
                       pltpu.VMEM((H2T,), jnp.int32),
                       pltpu.VMEM((4, 16), f32),
                       pltpu.VMEM_SHARED((NP, 16), f32),
                       pltpu.VMEM_SHARED((GP, 16), f32),
                       pltpu.VMEM_SHARED((GP, 16), f32),
                       pltpu.SemaphoreType.DMA,
                       pltpu.SemaphoreType.DMA],
    )(edge_index, zt, dinvb, batchp, b2q, zeros_n)

    out = pl.pallas_call(
        _tc_head,
        out_shape=jax.ShapeDtypeStruct((G, 1), f32),
    )(poolp, cntp, *wf1c, bf1r, Wf2, bf2r, Wf3, bf3r)

    return out
